# baseline (speedup 1.0000x reference)
import functools

import jax
import jax.numpy as jnp
from jax import lax
from jax.experimental import pallas as pl
from jax.experimental.pallas import tpu as pltpu
from jax.experimental.pallas import tpu_sc as plsc


@functools.lru_cache(maxsize=None)
def _build(V, S_s, S_m, D, R, cs):
    info = plsc.get_sparse_core_info()
    nc, ns = info.num_cores, info.num_subcores
    nw = nc * ns
    rpw = R // nw
    n_chunks = rpw // cs
    mesh = plsc.VectorSubcoreMesh(core_axis_name="c", subcore_axis_name="s")

    @functools.partial(
        pl.kernel,
        mesh=mesh,
        out_type=jax.ShapeDtypeStruct((R, S_m + S_s, D), jnp.float32),
        scratch_types=[
            pltpu.VMEM((n_chunks, cs), jnp.int32),
            pltpu.VMEM((cs, S_m + S_s, D), jnp.float32),
            pltpu.VMEM((cs, S_m + S_s, D), jnp.float32),
            pltpu.SemaphoreType.DMA,
            pltpu.SemaphoreType.DMA,
            pltpu.SemaphoreType.DMA,
            pltpu.SemaphoreType.DMA,
        ],
    )
    def k(market_hbm, ids2d_hbm, table_hbm, out_hbm,
          idx_v, c0, c1, gs0, gs1, ws0, ws1):
        wid = lax.axis_index("s") * nc + lax.axis_index("c")
        base = wid * rpw
        combo = (c0, c1)
        gsem = (gs0, gs1)
        wsem = (ws0, ws1)

        pltpu.sync_copy(ids2d_hbm.at[pl.ds(wid * n_chunks, n_chunks), :], idx_v)
        mfill = []
        for bb in (0, 1):
            for i in range(cs):
                mfill.append(pltpu.async_copy(
                    market_hbm, combo[bb].at[i, pl.ds(0, S_m), :], gsem[bb]))

        def start_gather(c, b):
            return pltpu.async_copy(
                table_hbm.at[idx_v.at[c]],
                combo[b].at[:, pl.ds(S_m, S_s), :], gsem[b])

        for d in mfill:
            d.wait()
        g = [start_gather(0, 0), None]
        w = [None, None]
        for c in range(n_chunks):
            b = c & 1
            nb = 1 - b
            if c + 1 < n_chunks:
                if c >= 1:
                    w[nb].wait()
                g[nb] = start_gather(c + 1, nb)
            row0 = base + c * cs
            g[b].wait()
            w[b] = pltpu.async_copy(combo[b], out_hbm.at[pl.ds(row0, cs)],
                                    wsem[b])
        w[(n_chunks - 1) & 1].wait()

    return k


def kernel(market_memory, symbol_memory, symbol_ids, batch_size, num_symbols):
    S_m, D = market_memory.shape
    V, S_s, _ = symbol_memory.shape
    b, n = symbol_ids.shape
    R = b * n
    cs = 16
    ids2d = symbol_ids.reshape(R // cs, cs).astype(jnp.int32)
    k = _build(V, S_s, S_m, D, R, cs)
    out = k(market_memory, ids2d, symbol_memory)
    return out.reshape(b, n, S_m + S_s, D)


# speedup vs baseline: 1.4184x; 1.4184x over previous
import functools

import jax
import jax.numpy as jnp
from jax import lax
from jax.experimental import pallas as pl
from jax.experimental.pallas import tpu as pltpu
from jax.experimental.pallas import tpu_sc as plsc


@functools.lru_cache(maxsize=None)
def _build(V, S_s, S_m, D, R, cs, mrep):
    info = plsc.get_sparse_core_info()
    nc, ns = info.num_cores, info.num_subcores
    nw = nc * ns
    rpw = R // nw
    n_chunks = rpw // cs
    mesh = plsc.VectorSubcoreMesh(core_axis_name="c", subcore_axis_name="s")

    @functools.partial(
        pl.kernel,
        mesh=mesh,
        out_type=jax.ShapeDtypeStruct((R, S_m + S_s, D), jnp.float32),
        scratch_types=[
            pltpu.VMEM((n_chunks, cs), jnp.int32),
            pltpu.VMEM((cs, S_s, D), jnp.float32),
            pltpu.VMEM((cs, S_s, D), jnp.float32),
            pltpu.VMEM((cs, S_s, D), jnp.float32),
            pltpu.VMEM((mrep, S_m, D), jnp.float32),
            pltpu.SemaphoreType.DMA,
            pltpu.SemaphoreType.DMA,
            pltpu.SemaphoreType.DMA,
            pltpu.SemaphoreType.DMA,
            pltpu.SemaphoreType.DMA,
            pltpu.SemaphoreType.DMA,
            pltpu.SemaphoreType.DMA,
        ],
    )
    def k(market_hbm, ids2d_hbm, table_hbm, out_hbm,
          idx_v, g0, g1, g2, mk_v, gs0, gs1, gs2, ws0, ws1, ws2, msem):
        wid = lax.axis_index("s") * nc + lax.axis_index("c")
        base = wid * rpw
        gath_v = (g0, g1, g2)
        gsem = (gs0, gs1, gs2)
        wsem = (ws0, ws1, ws2)

        pltpu.sync_copy(ids2d_hbm.at[pl.ds(wid * n_chunks, n_chunks), :], idx_v)
        mfill = [pltpu.async_copy(market_hbm, mk_v.at[i], msem)
                 for i in range(mrep)]

        def start_gather(c, b):
            return pltpu.async_copy(table_hbm.at[idx_v.at[c]], gath_v[b],
                                    gsem[b])

        g = [start_gather(0, 0), start_gather(1, 1), None]
        w = [None, None, None]
        m = [[], [], []]
        for d in mfill:
            d.wait()
        for c in range(n_chunks):
            b = c % 3
            pb = (c + 2) % 3
            if c + 2 < n_chunks:
                if w[pb] is not None:
                    w[pb].wait()
                g[pb] = start_gather(c + 2, pb)
            row0 = base + c * cs
            for d in m[b]:
                d.wait()
            m[b] = []
            g[b].wait()
            w[b] = pltpu.async_copy(
                gath_v[b], out_hbm.at[pl.ds(row0, cs), pl.ds(S_m, S_s), :],
                wsem[b])
            for j in range(cs // mrep):
                m[b].append(pltpu.async_copy(
                    mk_v,
                    out_hbm.at[pl.ds(row0 + j * mrep, mrep), pl.ds(0, S_m), :],
                    msem))
        for b in range(3):
            if w[b] is not None:
                w[b].wait()
            for d in m[b]:
                d.wait()

    return k


def kernel(market_memory, symbol_memory, symbol_ids, batch_size, num_symbols):
    S_m, D = market_memory.shape
    V, S_s, _ = symbol_memory.shape
    b, n = symbol_ids.shape
    R = b * n
    cs = 32
    ids2d = symbol_ids.reshape(R // cs, cs).astype(jnp.int32)
    k = _build(V, S_s, S_m, D, R, cs, 8)
    out = k(market_memory, ids2d, symbol_memory)
    return out.reshape(b, n, S_m + S_s, D)
